# baseline (speedup 1.0000x reference)
import functools

import jax
import jax.numpy as jnp
from jax import lax
from jax.experimental import pallas as pl
from jax.experimental.pallas import tpu as pltpu
from jax.experimental.pallas import tpu_sc as plsc

_NUM_CORES = 2
_NUM_SUBCORES = 16
_LANES = 16


def _make_sc_kernel(B, N, H):
    nodes_per_sub = N // _NUM_SUBCORES
    b_per_core = B // _NUM_CORES
    chunks_per_row = H // _LANES

    mesh = plsc.VectorSubcoreMesh(
        core_axis_name="c", subcore_axis_name="s")

    @functools.partial(
        pl.kernel,
        out_type=jax.ShapeDtypeStruct((B, N, H), jnp.float32),
        mesh=mesh,
        scratch_types=[
            pltpu.VMEM((nodes_per_sub,), jnp.int32),
            pltpu.VMEM((nodes_per_sub,), jnp.int32),
            pltpu.VMEM((nodes_per_sub, H), jnp.float32),
            pltpu.VMEM((nodes_per_sub, H), jnp.float32),
            pltpu.SemaphoreType.DMA,
            pltpu.SemaphoreType.DMA,
        ],
    )
    def sc_kernel(in_deg, out_deg, w_in, w_out, out,
                  idx_in_v, idx_out_v, a_v, b_v, isem, wsem):
        c = lax.axis_index("c")
        s = lax.axis_index("s")
        node0 = s * nodes_per_sub
        b0 = c * b_per_core

        ci = pltpu.async_copy(
            in_deg.at[pl.ds(node0, nodes_per_sub)], idx_in_v, isem)
        co = pltpu.async_copy(
            out_deg.at[pl.ds(node0, nodes_per_sub)], idx_out_v, isem)
        ci.wait()
        cp_a = pltpu.async_copy(w_in.at[idx_in_v], a_v, isem)
        co.wait()
        cp_b = pltpu.async_copy(w_out.at[idx_out_v], b_v, isem)
        cp_a.wait()
        cp_b.wait()

        unroll = 8
        for j in range(nodes_per_sub):
            def add_body(k, _, j=j):
                for u in range(unroll):
                    sl = pl.ds((k * unroll + u) * _LANES, _LANES)
                    a_v[j, sl] = a_v[j, sl] + b_v[j, sl]
                return _
            lax.fori_loop(0, chunks_per_row // unroll, add_body, None)

        copies = []
        for i in range(b_per_core):
            copies.append(
                pltpu.async_copy(
                    a_v, out.at[b0 + i, pl.ds(node0, nodes_per_sub)], wsem))
        for cp in copies:
            cp.wait()

    return sc_kernel


@jax.jit
def kernel(x, in_degree, out_degree, W_in, W_out):
    B = x.shape[0]
    N = in_degree.shape[0]
    H = W_in.shape[1]
    return _make_sc_kernel(B, N, H)(in_degree, out_degree, W_in, W_out)


# speedup vs baseline: 1.0456x; 1.0456x over previous
import functools

import jax
import jax.numpy as jnp
from jax import lax
from jax.experimental import pallas as pl
from jax.experimental.pallas import tpu as pltpu
from jax.experimental.pallas import tpu_sc as plsc

_NUM_CORES = 2
_NUM_SUBCORES = 16
_LANES = 16


def _make_sc_kernel(B, N, H):
    nodes_per_sub = N // _NUM_SUBCORES
    b_per_core = B // _NUM_CORES
    chunks_per_row = H // _LANES

    mesh = plsc.VectorSubcoreMesh(
        core_axis_name="c", subcore_axis_name="s")

    @functools.partial(
        pl.kernel,
        out_type=jax.ShapeDtypeStruct((B, N, H), jnp.float32),
        mesh=mesh,
        scratch_types=[
            pltpu.VMEM((nodes_per_sub,), jnp.int32),
            pltpu.VMEM((nodes_per_sub,), jnp.int32),
            pltpu.VMEM((nodes_per_sub, H), jnp.float32),
            pltpu.VMEM((nodes_per_sub, H), jnp.float32),
            pltpu.SemaphoreType.DMA,
            pltpu.SemaphoreType.DMA,
        ],
    )
    def sc_kernel(in_deg, out_deg, w_in, w_out, out,
                  idx_in_v, idx_out_v, a_v, b_v, isem, wsem):
        c = lax.axis_index("c")
        s = lax.axis_index("s")
        node0 = s * nodes_per_sub
        b0 = c * b_per_core

        ci = pltpu.async_copy(
            in_deg.at[pl.ds(node0, nodes_per_sub)], idx_in_v, isem)
        co = pltpu.async_copy(
            out_deg.at[pl.ds(node0, nodes_per_sub)], idx_out_v, isem)
        ci.wait()
        co.wait()

        cp_a = pltpu.async_copy(w_in.at[idx_in_v], a_v, isem)
        cp_b = pltpu.async_copy(w_out.at[idx_out_v], b_v, isem)
        cp_a.wait()
        cp_b.wait()

        unroll = 4
        for j in range(nodes_per_sub):
            def add_body(k, _, j=j):
                for u in range(unroll):
                    sl = pl.ds((k * unroll + u) * _LANES, _LANES)
                    a_v[j, sl] = a_v[j, sl] + b_v[j, sl]
                return _
            lax.fori_loop(0, chunks_per_row // unroll, add_body, None)

        copies = []
        for i in range(b_per_core):
            copies.append(
                pltpu.async_copy(
                    a_v, out.at[b0 + i, pl.ds(node0, nodes_per_sub)], wsem))
        for cp in copies:
            cp.wait()

    return sc_kernel


@jax.jit
def kernel(x, in_degree, out_degree, W_in, W_out):
    B = x.shape[0]
    N = in_degree.shape[0]
    H = W_in.shape[1]
    return _make_sc_kernel(B, N, H)(in_degree, out_degree, W_in, W_out)
